# baseline (speedup 1.0000x reference)
import functools

import jax
import jax.numpy as jnp
from jax import lax
from jax.experimental import pallas as pl
from jax.experimental.pallas import tpu as pltpu
from jax.experimental.pallas import tpu_sc as plsc

N = 10000
E = 320000
G = 256

NC = 2
NS = 16
NW = NC * NS
CH = 128
NCHUNK = E // CH
ITERS = (NCHUNK + NW - 1) // NW
PH = 40
NCHUNK_PAD = 2560
RPT = N // NS
DEGF = 16

BLK = 5000
NBLK = N // BLK


def _sc_mesh():
    return plsc.VectorSubcoreMesh(core_axis_name="c", subcore_axis_name="s")


_SC_PARAMS = pltpu.CompilerParams(use_tc_tiling_on_sc=False)


def _deg_body(dst2d_hbm, ones_hbm, zeros_hbm, out_hbm, didx, ones_v, acc, sem):
    c = lax.axis_index("c")
    s = lax.axis_index("s")
    w = c * NS + s
    r0 = s * RPT
    c0 = w * NCHUNK // NW
    n_w = (w + 1) * NCHUNK // NW - c0
    pltpu.sync_copy(dst2d_hbm.at[pl.ds(c0, ITERS)], didx)
    pltpu.sync_copy(ones_hbm, ones_v)
    pltpu.sync_copy(zeros_hbm, acc.at[pl.ds(r0, RPT)])
    plsc.subcore_barrier()

    LAG = 4

    def body(i, _):
        @pl.when(i < n_w)
        def _():
            @pl.when(i >= LAG)
            def _():
                pltpu.make_async_copy(ones_v, acc.at[didx.at[0]], sem).wait()

            pltpu.async_copy(ones_v, acc.at[didx.at[i]], sem, add=True)

        return 0

    lax.fori_loop(0, ITERS, body, 0)
    for _ in range(LAG):
        pltpu.make_async_copy(ones_v, acc.at[didx.at[0]], sem).wait()
    plsc.subcore_barrier()
    pltpu.sync_copy(acc.at[pl.ds(r0, RPT)], out_hbm.at[c, pl.ds(r0, RPT), :])


def _make_deg_kernel():
    return functools.partial(
        pl.kernel,
        out_type=jax.ShapeDtypeStruct((NC, N, DEGF), jnp.float32),
        mesh=_sc_mesh(),
        compiler_params=_SC_PARAMS,
        scratch_types=[
            pltpu.VMEM((ITERS, CH), jnp.int32),
            pltpu.VMEM((CH, DEGF), jnp.float32),
            pltpu.VMEM_SHARED((N, DEGF), jnp.float32),
            pltpu.SemaphoreType.DMA,
        ],
    )(_deg_body)


def _make_agg_body(nbuf, ch, ph):
    nchunk = E // ch
    iters = (nchunk + NW - 1) // NW
    npass = (iters + ph - 1) // ph

    def body(xs_hbm, src2d_hbm, dst2d_hbm, zeros_hbm, out_hbm, *scr):
        sidx, didx = scr[0], scr[1]
        rows = scr[2:2 + nbuf]
        acc = scr[2 + nbuf]
        gsems = scr[3 + nbuf:3 + 2 * nbuf]
        ssems = scr[3 + 2 * nbuf:3 + 3 * nbuf]
        c = lax.axis_index("c")
        s = lax.axis_index("s")
        w = c * NS + s
        r0 = s * RPT
        c0 = w * nchunk // NW
        n_w = (w + 1) * nchunk // NW - c0

        pltpu.sync_copy(zeros_hbm, acc.at[pl.ds(r0, RPT)])
        plsc.subcore_barrier()

        for p in range(npass):
            rem = jnp.minimum(n_w - ph * p, ph)
            pltpu.sync_copy(src2d_hbm.at[pl.ds(c0 + ph * p, ph)], sidx)
            pltpu.sync_copy(dst2d_hbm.at[pl.ds(c0 + ph * p, ph)], didx)

            def group(t, _, rem=rem):
                for k in range(nbuf):
                    j = nbuf * t + k

                    @pl.when(j < rem)
                    def _(j=j, k=k):
                        @pl.when(t >= 1)
                        def _(k=k):
                            pltpu.make_async_copy(
                                rows[k], acc.at[didx.at[0]], ssems[k]).wait()

                        pltpu.async_copy(
                            xs_hbm.at[sidx.at[j]], rows[k], gsems[k])

                for k in range(nbuf):
                    j = nbuf * t + k

                    @pl.when(j < rem)
                    def _(j=j, k=k):
                        pltpu.make_async_copy(
                            xs_hbm.at[sidx.at[j]], rows[k], gsems[k]).wait()
                        pltpu.async_copy(
                            rows[k], acc.at[didx.at[j]], ssems[k], add=True)

                return 0

            lax.fori_loop(0, ph // nbuf, group, 0)
            for k in range(nbuf):
                pltpu.make_async_copy(
                    rows[k], acc.at[didx.at[0]], ssems[k]).wait()
        plsc.subcore_barrier()
        pltpu.sync_copy(acc.at[pl.ds(r0, RPT)], out_hbm.at[c, pl.ds(r0, RPT), :])

    return body


def _make_agg_kernel(F, nbuf, ch, ph):
    return functools.partial(
        pl.kernel,
        out_type=jax.ShapeDtypeStruct((NC, N, F), jnp.float32),
        mesh=_sc_mesh(),
        compiler_params=_SC_PARAMS,
        scratch_types=(
            [pltpu.VMEM((ph, ch), jnp.int32),
             pltpu.VMEM((ph, ch), jnp.int32)]
            + [pltpu.VMEM((ch, F), jnp.float32) for _ in range(nbuf)]
            + [pltpu.VMEM_SHARED((N, F), jnp.float32)]
            + [pltpu.SemaphoreType.DMA for _ in range(2 * nbuf)]
        ),
    )(_make_agg_body(nbuf, ch, ph))


def _dinv_blk(degp_ref):
    deg = degp_ref[0, :, 0:1] + degp_ref[1, :, 0:1] + 1.0
    return lax.rsqrt(deg)


def _mm1_body(x_ref, w1_ref, degp_ref, xs1_ref):
    dinv = _dinv_blk(degp_ref)
    xw = jnp.dot(x_ref[...], w1_ref[...], preferred_element_type=jnp.float32)
    xs1_ref[...] = dinv * xw


def _mid_body(accp_ref, xs1_ref, degp_ref, w2_ref, b1_ref, xs2_ref):
    dinv = _dinv_blk(degp_ref)
    agg = accp_ref[0] + accp_ref[1] + xs1_ref[...]
    h1 = jnp.maximum(dinv * agg + b1_ref[...], 0.0)
    xw = jnp.dot(h1, w2_ref[...], preferred_element_type=jnp.float32)
    xs2_ref[...] = dinv * xw


def _fin_body(accp_ref, xs2_ref, degp_ref, b2_ref, batch_ref, wf_ref, bf_ref,
              out_ref, sums_ref, cnt_ref):
    i = pl.program_id(0)

    @pl.when(i == 0)
    def _():
        sums_ref[...] = jnp.zeros_like(sums_ref)
        cnt_ref[...] = jnp.zeros_like(cnt_ref)

    dinv = _dinv_blk(degp_ref)
    agg = accp_ref[0] + accp_ref[1] + xs2_ref[...]
    h2 = jnp.maximum(dinv * agg + b2_ref[...], 0.0)

    gids = lax.broadcasted_iota(jnp.int32, (BLK, G), 1)
    oh = (batch_ref[...] == gids).astype(jnp.float32)
    dn = (((0,), (0,)), ((), ()))
    sums_ref[...] += lax.dot_general(oh, h2, dn,
                                     preferred_element_type=jnp.float32)
    cnt_ref[...] += lax.dot_general(oh, jnp.ones((BLK, 1), jnp.float32), dn,
                                    preferred_element_type=jnp.float32)

    @pl.when(i == NBLK - 1)
    def _():
        pooled = sums_ref[...] / jnp.maximum(cnt_ref[...], 1.0)
        out_ref[...] = jnp.dot(pooled, wf_ref[...],
                               preferred_element_type=jnp.float32) + bf_ref[...]


def _mm1_call(x, W1, degp):
    return pl.pallas_call(
        _mm1_body,
        grid=(NBLK,),
        in_specs=[
            pl.BlockSpec((BLK, 128), lambda i: (i, 0)),
            pl.BlockSpec((128, 64), lambda i: (0, 0)),
            pl.BlockSpec((NC, BLK, DEGF), lambda i: (0, i, 0)),
        ],
        out_specs=pl.BlockSpec((BLK, 64), lambda i: (i, 0)),
        out_shape=jax.ShapeDtypeStruct((N, 64), jnp.float32),
    )(x, W1, degp)


def _mid_call(accp1, xs1, degp, W2, b1):
    return pl.pallas_call(
        _mid_body,
        grid=(NBLK,),
        in_specs=[
            pl.BlockSpec((NC, BLK, 64), lambda i: (0, i, 0)),
            pl.BlockSpec((BLK, 64), lambda i: (i, 0)),
            pl.BlockSpec((NC, BLK, DEGF), lambda i: (0, i, 0)),
            pl.BlockSpec((64, 128), lambda i: (0, 0)),
            pl.BlockSpec((1, 64), lambda i: (0, 0)),
        ],
        out_specs=pl.BlockSpec((BLK, 128), lambda i: (i, 0)),
        out_shape=jax.ShapeDtypeStruct((N, 128), jnp.float32),
    )(accp1, xs1, degp, W2, b1)


def _fin_call(accp2, xs2, degp, b2, batch2d, Wf, bf):
    return pl.pallas_call(
        _fin_body,
        grid=(NBLK,),
        in_specs=[
            pl.BlockSpec((NC, BLK, 128), lambda i: (0, i, 0)),
            pl.BlockSpec((BLK, 128), lambda i: (i, 0)),
            pl.BlockSpec((NC, BLK, DEGF), lambda i: (0, i, 0)),
            pl.BlockSpec((1, 128), lambda i: (0, 0)),
            pl.BlockSpec((BLK, 1), lambda i: (i, 0)),
            pl.BlockSpec((128, 128), lambda i: (0, 0)),
            pl.BlockSpec((1, 128), lambda i: (0, 0)),
        ],
        out_specs=pl.BlockSpec((G, 128), lambda i: (0, 0)),
        out_shape=jax.ShapeDtypeStruct((G, 128), jnp.float32),
        scratch_shapes=[
            pltpu.VMEM((G, 128), jnp.float32),
            pltpu.VMEM((G, 1), jnp.float32),
        ],
    )(accp2, xs2, degp, b2, batch2d, Wf, bf)


def kernel(x, edge_index, batch, W1, b1, W2, b2, Wf, bf):
    def edges2d(vec, ch, padrows):
        nchunk = E // ch
        return jnp.concatenate(
            [vec.reshape(nchunk, ch), jnp.zeros((padrows, ch), jnp.int32)])

    src2d = edges2d(edge_index[0], CH, NCHUNK_PAD - NCHUNK)
    dst2d = edges2d(edge_index[1], CH, NCHUNK_PAD - NCHUNK)
    src2d80 = edges2d(edge_index[0], 80, 32)
    dst2d80 = edges2d(edge_index[1], 80, 32)
    batch2d = batch.reshape(N, 1)
    b1r = b1.reshape(1, 64)
    b2r = b2.reshape(1, 128)
    bfr = bf.reshape(1, 128)

    ones_rows = jnp.ones((CH, DEGF), jnp.float32)
    zdeg = jnp.zeros((RPT, DEGF), jnp.float32)
    z64 = jnp.zeros((RPT, 64), jnp.float32)
    z128 = jnp.zeros((RPT, 128), jnp.float32)

    degp = _make_deg_kernel()(dst2d, ones_rows, zdeg)
    xs1 = _mm1_call(x, W1, degp)
    accp1 = _make_agg_kernel(64, 8, CH, PH)(xs1, src2d, dst2d, z64)
    xs2 = _mid_call(accp1, xs1, degp, W2, b1r)
    accp2 = _make_agg_kernel(128, 4, 80, 32)(xs2, src2d80, dst2d80, z128)
    return _fin_call(accp2, xs2, degp, b2r, batch2d, Wf, bfr)


# speedup vs baseline: 1.0071x; 1.0071x over previous
import functools

import jax
import jax.numpy as jnp
from jax import lax
from jax.experimental import pallas as pl
from jax.experimental.pallas import tpu as pltpu
from jax.experimental.pallas import tpu_sc as plsc

N = 10000
E = 320000
G = 256

NC = 2
NS = 16
NW = NC * NS
CH = 128
NCHUNK = E // CH
ITERS = (NCHUNK + NW - 1) // NW
PH = 40
NCHUNK_PAD = 2560
RPT = N // NS
DEGF = 16

BLK = 5000
NBLK = N // BLK


def _sc_mesh():
    return plsc.VectorSubcoreMesh(core_axis_name="c", subcore_axis_name="s")


_SC_PARAMS = pltpu.CompilerParams(use_tc_tiling_on_sc=False)


def _deg_body(dst2d_hbm, ones_hbm, zeros_hbm, out_hbm, didx, ones_v, acc, sem):
    c = lax.axis_index("c")
    s = lax.axis_index("s")
    w = c * NS + s
    r0 = s * RPT
    c0 = w * NCHUNK // NW
    n_w = (w + 1) * NCHUNK // NW - c0
    pltpu.sync_copy(dst2d_hbm.at[pl.ds(c0, ITERS)], didx)
    pltpu.sync_copy(ones_hbm, ones_v)
    pltpu.sync_copy(zeros_hbm, acc.at[pl.ds(r0, RPT)])
    plsc.subcore_barrier()

    LAG = 4

    def body(i, _):
        @pl.when(i < n_w)
        def _():
            @pl.when(i >= LAG)
            def _():
                pltpu.make_async_copy(ones_v, acc.at[didx.at[0]], sem).wait()

            pltpu.async_copy(ones_v, acc.at[didx.at[i]], sem, add=True)

        return 0

    lax.fori_loop(0, ITERS, body, 0)
    for _ in range(LAG):
        pltpu.make_async_copy(ones_v, acc.at[didx.at[0]], sem).wait()
    plsc.subcore_barrier()
    pltpu.sync_copy(acc.at[pl.ds(r0, RPT)], out_hbm.at[c, pl.ds(r0, RPT), :])


def _make_deg_kernel():
    return functools.partial(
        pl.kernel,
        out_type=jax.ShapeDtypeStruct((NC, N, DEGF), jnp.float32),
        mesh=_sc_mesh(),
        compiler_params=_SC_PARAMS,
        scratch_types=[
            pltpu.VMEM((ITERS, CH), jnp.int32),
            pltpu.VMEM((CH, DEGF), jnp.float32),
            pltpu.VMEM_SHARED((N, DEGF), jnp.float32),
            pltpu.SemaphoreType.DMA,
        ],
    )(_deg_body)


def _make_agg_body(nbuf, ch, ph):
    nchunk = E // ch
    iters = (nchunk + NW - 1) // NW
    npass = (iters + ph - 1) // ph

    def body(xs_hbm, src2d_hbm, dst2d_hbm, zeros_hbm, out_hbm, *scr):
        sidx, didx = scr[0], scr[1]
        rows = scr[2:2 + nbuf]
        acc = scr[2 + nbuf]
        gsems = scr[3 + nbuf:3 + 2 * nbuf]
        ssems = scr[3 + 2 * nbuf:3 + 3 * nbuf]
        c = lax.axis_index("c")
        s = lax.axis_index("s")
        w = c * NS + s
        r0 = s * RPT
        c0 = w * nchunk // NW
        n_w = (w + 1) * nchunk // NW - c0

        pltpu.sync_copy(zeros_hbm, acc.at[pl.ds(r0, RPT)])
        plsc.subcore_barrier()

        for p in range(npass):
            rem = jnp.minimum(n_w - ph * p, ph)
            pltpu.sync_copy(src2d_hbm.at[pl.ds(c0 + ph * p, ph)], sidx)
            pltpu.sync_copy(dst2d_hbm.at[pl.ds(c0 + ph * p, ph)], didx)

            def group(t, _, rem=rem):
                for k in range(nbuf):
                    j = nbuf * t + k

                    @pl.when(j < rem)
                    def _(j=j, k=k):
                        @pl.when(t >= 1)
                        def _(k=k):
                            pltpu.make_async_copy(
                                rows[k], acc.at[didx.at[0]], ssems[k]).wait()

                        pltpu.async_copy(
                            xs_hbm.at[sidx.at[j]], rows[k], gsems[k])

                for k in range(nbuf):
                    j = nbuf * t + k

                    @pl.when(j < rem)
                    def _(j=j, k=k):
                        pltpu.make_async_copy(
                            xs_hbm.at[sidx.at[j]], rows[k], gsems[k]).wait()
                        pltpu.async_copy(
                            rows[k], acc.at[didx.at[j]], ssems[k], add=True)

                return 0

            lax.fori_loop(0, ph // nbuf, group, 0)
            for k in range(nbuf):
                pltpu.make_async_copy(
                    rows[k], acc.at[didx.at[0]], ssems[k]).wait()
        plsc.subcore_barrier()
        pltpu.sync_copy(acc.at[pl.ds(r0, RPT)], out_hbm.at[c, pl.ds(r0, RPT), :])

    return body


def _make_agg_kernel(F, nbuf, ch, ph):
    return functools.partial(
        pl.kernel,
        out_type=jax.ShapeDtypeStruct((NC, N, F), jnp.float32),
        mesh=_sc_mesh(),
        compiler_params=_SC_PARAMS,
        scratch_types=(
            [pltpu.VMEM((ph, ch), jnp.int32),
             pltpu.VMEM((ph, ch), jnp.int32)]
            + [pltpu.VMEM((ch, F), jnp.float32) for _ in range(nbuf)]
            + [pltpu.VMEM_SHARED((N, F), jnp.float32)]
            + [pltpu.SemaphoreType.DMA for _ in range(2 * nbuf)]
        ),
    )(_make_agg_body(nbuf, ch, ph))


def _dinv_blk(degp_ref):
    deg = degp_ref[0, :, 0:1] + degp_ref[1, :, 0:1] + 1.0
    return lax.rsqrt(deg)


def _mm1_body(x_ref, w1_ref, degp_ref, xs1_ref):
    dinv = _dinv_blk(degp_ref)
    xw = jnp.dot(x_ref[...], w1_ref[...], preferred_element_type=jnp.float32)
    xs1_ref[...] = dinv * xw


def _mid_body(accp_ref, xs1_ref, degp_ref, w2_ref, b1_ref, xs2_ref):
    dinv = _dinv_blk(degp_ref)
    agg = accp_ref[0] + accp_ref[1] + xs1_ref[...]
    h1 = jnp.maximum(dinv * agg + b1_ref[...], 0.0)
    xw = jnp.dot(h1, w2_ref[...], preferred_element_type=jnp.float32)
    xs2_ref[...] = dinv * xw


def _fin_body(accp_ref, xs2_ref, degp_ref, b2_ref, batch_ref, wf_ref, bf_ref,
              out_ref, sums_ref, cnt_ref):
    i = pl.program_id(0)

    @pl.when(i == 0)
    def _():
        sums_ref[...] = jnp.zeros_like(sums_ref)
        cnt_ref[...] = jnp.zeros_like(cnt_ref)

    dinv = _dinv_blk(degp_ref)
    agg = accp_ref[0] + accp_ref[1] + xs2_ref[...]
    h2 = jnp.maximum(dinv * agg + b2_ref[...], 0.0)

    gids = lax.broadcasted_iota(jnp.int32, (BLK, G), 1)
    oh = (batch_ref[...] == gids).astype(jnp.float32)
    dn = (((0,), (0,)), ((), ()))
    sums_ref[...] += lax.dot_general(oh, h2, dn,
                                     preferred_element_type=jnp.float32)
    cnt_ref[...] += lax.dot_general(oh, jnp.ones((BLK, 1), jnp.float32), dn,
                                    preferred_element_type=jnp.float32)

    @pl.when(i == NBLK - 1)
    def _():
        pooled = sums_ref[...] / jnp.maximum(cnt_ref[...], 1.0)
        out_ref[...] = jnp.dot(pooled, wf_ref[...],
                               preferred_element_type=jnp.float32) + bf_ref[...]


def _mm1_call(x, W1, degp):
    return pl.pallas_call(
        _mm1_body,
        grid=(NBLK,),
        in_specs=[
            pl.BlockSpec((BLK, 128), lambda i: (i, 0)),
            pl.BlockSpec((128, 64), lambda i: (0, 0)),
            pl.BlockSpec((NC, BLK, DEGF), lambda i: (0, i, 0)),
        ],
        out_specs=pl.BlockSpec((BLK, 64), lambda i: (i, 0)),
        out_shape=jax.ShapeDtypeStruct((N, 64), jnp.float32),
    )(x, W1, degp)


def _mid_call(accp1, xs1, degp, W2, b1):
    return pl.pallas_call(
        _mid_body,
        grid=(NBLK,),
        in_specs=[
            pl.BlockSpec((NC, BLK, 64), lambda i: (0, i, 0)),
            pl.BlockSpec((BLK, 64), lambda i: (i, 0)),
            pl.BlockSpec((NC, BLK, DEGF), lambda i: (0, i, 0)),
            pl.BlockSpec((64, 128), lambda i: (0, 0)),
            pl.BlockSpec((1, 64), lambda i: (0, 0)),
        ],
        out_specs=pl.BlockSpec((BLK, 128), lambda i: (i, 0)),
        out_shape=jax.ShapeDtypeStruct((N, 128), jnp.float32),
    )(accp1, xs1, degp, W2, b1)


def _fin_call(accp2, xs2, degp, b2, batch2d, Wf, bf):
    return pl.pallas_call(
        _fin_body,
        grid=(NBLK,),
        in_specs=[
            pl.BlockSpec((NC, BLK, 128), lambda i: (0, i, 0)),
            pl.BlockSpec((BLK, 128), lambda i: (i, 0)),
            pl.BlockSpec((NC, BLK, DEGF), lambda i: (0, i, 0)),
            pl.BlockSpec((1, 128), lambda i: (0, 0)),
            pl.BlockSpec((BLK, 1), lambda i: (i, 0)),
            pl.BlockSpec((128, 128), lambda i: (0, 0)),
            pl.BlockSpec((1, 128), lambda i: (0, 0)),
        ],
        out_specs=pl.BlockSpec((G, 128), lambda i: (0, 0)),
        out_shape=jax.ShapeDtypeStruct((G, 128), jnp.float32),
        scratch_shapes=[
            pltpu.VMEM((G, 128), jnp.float32),
            pltpu.VMEM((G, 1), jnp.float32),
        ],
    )(accp2, xs2, degp, b2, batch2d, Wf, bf)


def kernel(x, edge_index, batch, W1, b1, W2, b2, Wf, bf):
    def edges2d(vec, ch, padrows):
        nchunk = E // ch
        return jnp.concatenate(
            [vec.reshape(nchunk, ch), jnp.zeros((padrows, ch), jnp.int32)])

    src2d = edges2d(edge_index[0], CH, NCHUNK_PAD - NCHUNK)
    dst2d = edges2d(edge_index[1], CH, NCHUNK_PAD - NCHUNK)
    src2d80 = edges2d(edge_index[0], 80, 32)
    dst2d80 = edges2d(edge_index[1], 80, 32)
    batch2d = batch.reshape(N, 1)
    b1r = b1.reshape(1, 64)
    b2r = b2.reshape(1, 128)
    bfr = bf.reshape(1, 128)

    ones_rows = jnp.ones((CH, DEGF), jnp.float32)
    zdeg = jnp.zeros((RPT, DEGF), jnp.float32)
    z64 = jnp.zeros((RPT, 64), jnp.float32)
    z128 = jnp.zeros((RPT, 128), jnp.float32)

    degp = _make_deg_kernel()(dst2d, ones_rows, zdeg)
    xs1 = _mm1_call(x, W1, degp)
    accp1 = _make_agg_kernel(64, 8, CH, 80)(xs1, src2d, dst2d, z64)
    xs2 = _mid_call(accp1, xs1, degp, W2, b1r)
    accp2 = _make_agg_kernel(128, 4, 80, 32)(xs2, src2d80, dst2d80, z128)
    return _fin_call(accp2, xs2, degp, b2r, batch2d, Wf, bfr)


# speedup vs baseline: 1.0155x; 1.0083x over previous
import functools

import jax
import jax.numpy as jnp
from jax import lax
from jax.experimental import pallas as pl
from jax.experimental.pallas import tpu as pltpu
from jax.experimental.pallas import tpu_sc as plsc

N = 10000
E = 320000
G = 256

NC = 2
NS = 16
NW = NC * NS
CH = 128
NCHUNK = E // CH
ITERS = (NCHUNK + NW - 1) // NW
PH = 40
NCHUNK_PAD = 2560
RPT = N // NS
DEGF = 16

BLK = 5000
NBLK = N // BLK


def _sc_mesh():
    return plsc.VectorSubcoreMesh(core_axis_name="c", subcore_axis_name="s")


_SC_PARAMS = pltpu.CompilerParams(use_tc_tiling_on_sc=False)


def _deg_body(dst2d_hbm, ones_hbm, zeros_hbm, out_hbm, didx, ones_v, acc, sem):
    c = lax.axis_index("c")
    s = lax.axis_index("s")
    w = c * NS + s
    r0 = s * RPT
    c0 = w * NCHUNK // NW
    n_w = (w + 1) * NCHUNK // NW - c0
    pltpu.sync_copy(dst2d_hbm.at[pl.ds(c0, ITERS)], didx)
    pltpu.sync_copy(ones_hbm, ones_v)
    pltpu.sync_copy(zeros_hbm, acc.at[pl.ds(r0, RPT)])
    plsc.subcore_barrier()

    LAG = 4

    def body(i, _):
        @pl.when(i < n_w)
        def _():
            @pl.when(i >= LAG)
            def _():
                pltpu.make_async_copy(ones_v, acc.at[didx.at[0]], sem).wait()

            pltpu.async_copy(ones_v, acc.at[didx.at[i]], sem, add=True)

        return 0

    lax.fori_loop(0, ITERS, body, 0)
    for _ in range(LAG):
        pltpu.make_async_copy(ones_v, acc.at[didx.at[0]], sem).wait()
    plsc.subcore_barrier()
    pltpu.sync_copy(acc.at[pl.ds(r0, RPT)], out_hbm.at[c, pl.ds(r0, RPT), :])


def _make_deg_kernel():
    return functools.partial(
        pl.kernel,
        out_type=jax.ShapeDtypeStruct((NC, N, DEGF), jnp.float32),
        mesh=_sc_mesh(),
        compiler_params=_SC_PARAMS,
        scratch_types=[
            pltpu.VMEM((ITERS, CH), jnp.int32),
            pltpu.VMEM((CH, DEGF), jnp.float32),
            pltpu.VMEM_SHARED((N, DEGF), jnp.float32),
            pltpu.SemaphoreType.DMA,
        ],
    )(_deg_body)


def _make_agg_body(nbuf, ch, ph):
    nchunk = E // ch
    iters = (nchunk + NW - 1) // NW
    npass = (iters + ph - 1) // ph

    def body(xs_hbm, src2d_hbm, dst2d_hbm, zeros_hbm, out_hbm, *scr):
        sidx, didx = scr[0], scr[1]
        rows = scr[2:2 + nbuf]
        acc = scr[2 + nbuf]
        gsems = scr[3 + nbuf:3 + 2 * nbuf]
        ssems = scr[3 + 2 * nbuf:3 + 3 * nbuf]
        c = lax.axis_index("c")
        s = lax.axis_index("s")
        w = c * NS + s
        r0 = s * RPT
        c0 = w * nchunk // NW
        n_w = (w + 1) * nchunk // NW - c0

        pltpu.sync_copy(zeros_hbm, acc.at[pl.ds(r0, RPT)])
        plsc.subcore_barrier()

        for p in range(npass):
            rem = jnp.minimum(n_w - ph * p, ph)
            pltpu.sync_copy(src2d_hbm.at[pl.ds(c0 + ph * p, ph)], sidx)
            pltpu.sync_copy(dst2d_hbm.at[pl.ds(c0 + ph * p, ph)], didx)

            def group(t, _, rem=rem):
                for k in range(nbuf):
                    j = nbuf * t + k

                    @pl.when(j < rem)
                    def _(j=j, k=k):
                        @pl.when(t >= 1)
                        def _(k=k):
                            pltpu.make_async_copy(
                                rows[k], acc.at[didx.at[0]], ssems[k]).wait()

                        pltpu.async_copy(
                            xs_hbm.at[sidx.at[j]], rows[k], gsems[k])

                for k in range(nbuf):
                    j = nbuf * t + k

                    @pl.when(j < rem)
                    def _(j=j, k=k):
                        pltpu.make_async_copy(
                            xs_hbm.at[sidx.at[j]], rows[k], gsems[k]).wait()
                        pltpu.async_copy(
                            rows[k], acc.at[didx.at[j]], ssems[k], add=True)

                return 0

            lax.fori_loop(0, ph // nbuf, group, 0)
            for k in range(nbuf):
                pltpu.make_async_copy(
                    rows[k], acc.at[didx.at[0]], ssems[k]).wait()
        plsc.subcore_barrier()
        pltpu.sync_copy(acc.at[pl.ds(r0, RPT)], out_hbm.at[c, pl.ds(r0, RPT), :])

    return body


def _make_agg_kernel(F, nbuf, ch, ph):
    return functools.partial(
        pl.kernel,
        out_type=jax.ShapeDtypeStruct((NC, N, F), jnp.float32),
        mesh=_sc_mesh(),
        compiler_params=_SC_PARAMS,
        scratch_types=(
            [pltpu.VMEM((ph, ch), jnp.int32),
             pltpu.VMEM((ph, ch), jnp.int32)]
            + [pltpu.VMEM((ch, F), jnp.float32) for _ in range(nbuf)]
            + [pltpu.VMEM_SHARED((N, F), jnp.float32)]
            + [pltpu.SemaphoreType.DMA for _ in range(2 * nbuf)]
        ),
    )(_make_agg_body(nbuf, ch, ph))


def _dinv_blk(degp_ref):
    deg = degp_ref[0, :, 0:1] + degp_ref[1, :, 0:1] + 1.0
    return lax.rsqrt(deg)


def _mm1_body(x_ref, w1_ref, degp_ref, xs1_ref):
    dinv = _dinv_blk(degp_ref)
    xw = jnp.dot(x_ref[...], w1_ref[...], preferred_element_type=jnp.float32)
    xs1_ref[...] = dinv * xw


def _mid_body(accp_ref, xs1_ref, degp_ref, w2_ref, b1_ref, xs2_ref):
    dinv = _dinv_blk(degp_ref)
    agg = accp_ref[0] + accp_ref[1] + xs1_ref[...]
    h1 = jnp.maximum(dinv * agg + b1_ref[...], 0.0)
    xw = jnp.dot(h1, w2_ref[...], preferred_element_type=jnp.float32)
    xs2_ref[...] = dinv * xw


def _fin_body(accp_ref, xs2_ref, degp_ref, b2_ref, batch_ref, wf_ref, bf_ref,
              out_ref, sums_ref, cnt_ref):
    i = pl.program_id(0)

    @pl.when(i == 0)
    def _():
        sums_ref[...] = jnp.zeros_like(sums_ref)
        cnt_ref[...] = jnp.zeros_like(cnt_ref)

    dinv = _dinv_blk(degp_ref)
    agg = accp_ref[0] + accp_ref[1] + xs2_ref[...]
    h2 = jnp.maximum(dinv * agg + b2_ref[...], 0.0)

    gids = lax.broadcasted_iota(jnp.int32, (BLK, G), 1)
    oh = (batch_ref[...] == gids).astype(jnp.float32)
    dn = (((0,), (0,)), ((), ()))
    sums_ref[...] += lax.dot_general(oh, h2, dn,
                                     preferred_element_type=jnp.float32)
    cnt_ref[...] += lax.dot_general(oh, jnp.ones((BLK, 1), jnp.float32), dn,
                                    preferred_element_type=jnp.float32)

    @pl.when(i == NBLK - 1)
    def _():
        pooled = sums_ref[...] / jnp.maximum(cnt_ref[...], 1.0)
        out_ref[...] = jnp.dot(pooled, wf_ref[...],
                               preferred_element_type=jnp.float32) + bf_ref[...]


def _mm1_call(x, W1, degp):
    return pl.pallas_call(
        _mm1_body,
        grid=(NBLK,),
        in_specs=[
            pl.BlockSpec((BLK, 128), lambda i: (i, 0)),
            pl.BlockSpec((128, 64), lambda i: (0, 0)),
            pl.BlockSpec((NC, BLK, DEGF), lambda i: (0, i, 0)),
        ],
        out_specs=pl.BlockSpec((BLK, 64), lambda i: (i, 0)),
        out_shape=jax.ShapeDtypeStruct((N, 64), jnp.float32),
    )(x, W1, degp)


def _mid_call(accp1, xs1, degp, W2, b1):
    return pl.pallas_call(
        _mid_body,
        grid=(NBLK,),
        in_specs=[
            pl.BlockSpec((NC, BLK, 64), lambda i: (0, i, 0)),
            pl.BlockSpec((BLK, 64), lambda i: (i, 0)),
            pl.BlockSpec((NC, BLK, DEGF), lambda i: (0, i, 0)),
            pl.BlockSpec((64, 128), lambda i: (0, 0)),
            pl.BlockSpec((1, 64), lambda i: (0, 0)),
        ],
        out_specs=pl.BlockSpec((BLK, 128), lambda i: (i, 0)),
        out_shape=jax.ShapeDtypeStruct((N, 128), jnp.float32),
    )(accp1, xs1, degp, W2, b1)


def _fin_call(accp2, xs2, degp, b2, batch2d, Wf, bf):
    return pl.pallas_call(
        _fin_body,
        grid=(NBLK,),
        in_specs=[
            pl.BlockSpec((NC, BLK, 128), lambda i: (0, i, 0)),
            pl.BlockSpec((BLK, 128), lambda i: (i, 0)),
            pl.BlockSpec((NC, BLK, DEGF), lambda i: (0, i, 0)),
            pl.BlockSpec((1, 128), lambda i: (0, 0)),
            pl.BlockSpec((BLK, 1), lambda i: (i, 0)),
            pl.BlockSpec((128, 128), lambda i: (0, 0)),
            pl.BlockSpec((1, 128), lambda i: (0, 0)),
        ],
        out_specs=pl.BlockSpec((G, 128), lambda i: (0, 0)),
        out_shape=jax.ShapeDtypeStruct((G, 128), jnp.float32),
        scratch_shapes=[
            pltpu.VMEM((G, 128), jnp.float32),
            pltpu.VMEM((G, 1), jnp.float32),
        ],
    )(accp2, xs2, degp, b2, batch2d, Wf, bf)


def kernel(x, edge_index, batch, W1, b1, W2, b2, Wf, bf):
    def edges2d(vec, ch, padrows):
        nchunk = E // ch
        return jnp.concatenate(
            [vec.reshape(nchunk, ch), jnp.zeros((padrows, ch), jnp.int32)])

    src2d = edges2d(edge_index[0], CH, NCHUNK_PAD - NCHUNK)
    dst2d = edges2d(edge_index[1], CH, NCHUNK_PAD - NCHUNK)
    src2d80 = edges2d(edge_index[0], 80, 64)
    dst2d80 = edges2d(edge_index[1], 80, 64)
    batch2d = batch.reshape(N, 1)
    b1r = b1.reshape(1, 64)
    b2r = b2.reshape(1, 128)
    bfr = bf.reshape(1, 128)

    ones_rows = jnp.ones((CH, DEGF), jnp.float32)
    zdeg = jnp.zeros((RPT, DEGF), jnp.float32)
    z64 = jnp.zeros((RPT, 64), jnp.float32)
    z128 = jnp.zeros((RPT, 128), jnp.float32)

    degp = _make_deg_kernel()(dst2d, ones_rows, zdeg)
    xs1 = _mm1_call(x, W1, degp)
    accp1 = _make_agg_kernel(64, 8, CH, 80)(xs1, src2d, dst2d, z64)
    xs2 = _mid_call(accp1, xs1, degp, W2, b1r)
    accp2 = _make_agg_kernel(128, 4, 80, 60)(xs2, src2d80, dst2d80, z128)
    return _fin_call(accp2, xs2, degp, b2r, batch2d, Wf, bfr)
